# baseline (speedup 1.0000x reference)
import jax
import jax.numpy as jnp
from jax.experimental import pallas as pl

_NCLS = 128
_BB = 128


def _body(x_ref, o_ref):
    idx = x_ref[...]
    iota = jax.lax.broadcasted_iota(jnp.int32, (1, 1, _NCLS), 2)
    o_ref[...] = (idx[:, :, None] == iota).astype(jnp.float32)


def kernel(inputs):
    x = inputs.astype(jnp.int32)
    b, d = x.shape
    out = pl.pallas_call(
        _body,
        grid=(b // _BB,),
        in_specs=[pl.BlockSpec((_BB, d), lambda i: (i, 0))],
        out_specs=pl.BlockSpec((_BB, d, _NCLS), lambda i: (i, 0, 0)),
        out_shape=jax.ShapeDtypeStruct((b, d, _NCLS), jnp.float32),
    )(x)
    return out.reshape(b, d * _NCLS)


# speedup vs baseline: 1.4168x; 1.4168x over previous
import functools

import jax
import jax.numpy as jnp
from jax import lax
from jax.experimental import pallas as pl
from jax.experimental.pallas import tpu as pltpu
from jax.experimental.pallas import tpu_sc as plsc

_B, _D, _C = 4096, 100, 128
_R = 8
_OFFS = (0, 16, 32, 48, 64, 80, 84)
_BUF = _R * _D * _C


def _make_sc_kernel():
    info = plsc.get_sparse_core_info()
    nc, ns = info.num_cores, info.num_subcores
    nw = nc * ns
    rows_w = _B // nw
    chunks = rows_w // _R
    mesh = plsc.VectorSubcoreMesh(core_axis_name="c", subcore_axis_name="s")

    @functools.partial(
        pl.kernel,
        mesh=mesh,
        out_type=jax.ShapeDtypeStruct((_B * _D * _C,), jnp.float32),
        scratch_types=[
            pltpu.VMEM((_R * _D,), jnp.int32),
            pltpu.VMEM((_BUF,), jnp.float32),
        ],
        compiler_params=pltpu.CompilerParams(needs_layout_passes=False),
    )
    def k(x_hbm, out_hbm, idx_v, buf_v):
        wid = lax.axis_index("s") * nc + lax.axis_index("c")
        zeros16 = jnp.zeros((16,), jnp.float32)
        ones16 = jnp.ones((16,), jnp.float32)
        lane = lax.broadcasted_iota(jnp.int32, (16,), 0) * _C

        def zbody(i, carry):
            base = i * 64
            for u in range(4):
                buf_v[pl.ds(base + u * 16, 16)] = zeros16
            return carry

        lax.fori_loop(0, _BUF // 64, zbody, 0)

        def scatter_all(val16):
            for r in range(_R):
                for off in _OFFS:
                    vals = idx_v[pl.ds(r * _D + off, 16)]
                    pos = vals + lane + (r * _D * _C + off * _C)
                    plsc.store_scatter(buf_v, [pos], val16)

        def chunk(g, carry):
            rowstart = wid * rows_w + g * _R
            pltpu.sync_copy(x_hbm.at[pl.ds(rowstart * _D, _R * _D)], idx_v)
            scatter_all(ones16)
            pltpu.sync_copy(buf_v, out_hbm.at[pl.ds(rowstart * _D * _C, _BUF)])
            scatter_all(zeros16)
            return carry

        lax.fori_loop(0, chunks, chunk, 0)

    return k


def kernel(inputs):
    x = inputs.astype(jnp.int32).reshape(-1)
    out = _make_sc_kernel()(x)
    return out.reshape(_B, _D * _C)


# speedup vs baseline: 1.4942x; 1.0546x over previous
import functools

import jax
import jax.numpy as jnp
from jax import lax
from jax.experimental import pallas as pl
from jax.experimental.pallas import tpu as pltpu
from jax.experimental.pallas import tpu_sc as plsc

_B, _D, _C = 4096, 100, 128
_R = 4
_OFFS = (0, 16, 32, 48, 64, 80, 84)
_CW = _R * _D
_CBUF = _R * _D * _C


def _make_sc_kernel():
    info = plsc.get_sparse_core_info()
    nc, ns = info.num_cores, info.num_subcores
    nw = nc * ns
    rows_w = _B // nw
    chunks = rows_w // _R
    mesh = plsc.VectorSubcoreMesh(core_axis_name="c", subcore_axis_name="s")

    @functools.partial(
        pl.kernel,
        mesh=mesh,
        out_type=jax.ShapeDtypeStruct((_B * _D * _C,), jnp.float32),
        scratch_types=[
            pltpu.VMEM((rows_w * _D,), jnp.int32),
            pltpu.VMEM((_CBUF,), jnp.float32),
            pltpu.VMEM((_CBUF,), jnp.float32),
            pltpu.SemaphoreType.DMA,
            pltpu.SemaphoreType.DMA,
        ],
        compiler_params=pltpu.CompilerParams(needs_layout_passes=False),
    )
    def k(x_hbm, out_hbm, idx_v, buf_a, buf_b, sem_a, sem_b):
        wid = lax.axis_index("s") * nc + lax.axis_index("c")
        zeros16 = jnp.zeros((16,), jnp.float32)
        ones16 = jnp.ones((16,), jnp.float32)
        lane = lax.broadcasted_iota(jnp.int32, (16,), 0) * _C

        pltpu.sync_copy(x_hbm.at[pl.ds(wid * rows_w * _D, rows_w * _D)], idx_v)

        def zbody(i, carry):
            base = i * 64
            for u in range(4):
                buf_a[pl.ds(base + u * 16, 16)] = zeros16
                buf_b[pl.ds(base + u * 16, 16)] = zeros16
            return carry

        lax.fori_loop(0, _CBUF // 64, zbody, 0)

        def scatter(buf, cbase, val16):
            for r in range(_R):
                for off in _OFFS:
                    vals = idx_v[pl.ds(cbase + r * _D + off, 16)]
                    pos = vals + lane + (r * _D * _C + off * _C)
                    plsc.store_scatter(buf, [pos], val16)

        def out_slice(i):
            return out_hbm.at[pl.ds(wid * rows_w * _D * _C + i * _CBUF, _CBUF)]

        def body(i, carry):
            for parity, buf, sem in ((0, buf_a, sem_a), (1, buf_b, sem_b)):

                @pl.when(lax.rem(i, 2) == parity)
                def _():
                    @pl.when(i >= 2)
                    def _():
                        pltpu.make_async_copy(buf, out_slice(i - 2), sem).wait()
                        scatter(buf, (i - 2) * _CW, zeros16)

                    scatter(buf, i * _CW, ones16)
                    pltpu.async_copy(buf, out_slice(i), sem)

            return carry

        lax.fori_loop(0, chunks, body, 0)
        pltpu.make_async_copy(buf_a, out_slice(chunks - 2), sem_a).wait()
        pltpu.make_async_copy(buf_b, out_slice(chunks - 1), sem_b).wait()

    return k


def kernel(inputs):
    x = inputs.astype(jnp.int32).reshape(-1)
    out = _make_sc_kernel()(x)
    return out.reshape(_B, _D * _C)


# speedup vs baseline: 4.5773x; 3.0635x over previous
import functools

import jax
import jax.numpy as jnp
from jax import lax
from jax.experimental import pallas as pl
from jax.experimental.pallas import tpu as pltpu
from jax.experimental.pallas import tpu_sc as plsc

_B, _D, _C = 4096, 100, 128
_R = 4
_OFFS = (0, 16, 32, 48, 64, 80, 84)
_CW = _R * _D


def _make_sc_kernel():
    info = plsc.get_sparse_core_info()
    nc, ns = info.num_cores, info.num_subcores
    nw = nc * ns
    rows_w = _B // nw
    chunks = rows_w // _R
    mesh = plsc.VectorSubcoreMesh(core_axis_name="c", subcore_axis_name="s")

    @functools.partial(
        pl.kernel,
        mesh=mesh,
        out_type=jax.ShapeDtypeStruct((_B, _D * _C), jnp.float32),
        scratch_types=[
            pltpu.VMEM((rows_w * _D,), jnp.int32),
            pltpu.VMEM((_R, _D * _C), jnp.float32),
            pltpu.VMEM((_R, _D * _C), jnp.float32),
            pltpu.SemaphoreType.DMA,
            pltpu.SemaphoreType.DMA,
        ],
        compiler_params=pltpu.CompilerParams(needs_layout_passes=False),
    )
    def k(x_hbm, out_hbm, idx_v, buf_a, buf_b, sem_a, sem_b):
        wid = lax.axis_index("s") * nc + lax.axis_index("c")
        zeros16 = jnp.zeros((16,), jnp.float32)
        ones16 = jnp.ones((16,), jnp.float32)
        lane = lax.broadcasted_iota(jnp.int32, (16,), 0) * _C

        pltpu.sync_copy(x_hbm.at[pl.ds(wid * rows_w * _D, rows_w * _D)], idx_v)

        def zbody(i, carry):
            base = i * 64
            for r in range(_R):
                for u in range(4):
                    buf_a[r, pl.ds(base + u * 16, 16)] = zeros16
                    buf_b[r, pl.ds(base + u * 16, 16)] = zeros16
            return carry

        lax.fori_loop(0, _D * _C // 64, zbody, 0)

        def scatter(buf, cbase, val16):
            for r in range(_R):
                rowv = jnp.full((16,), r, jnp.int32)
                for off in _OFFS:
                    vals = idx_v[pl.ds(cbase + r * _D + off, 16)]
                    cols = vals + lane + off * _C
                    plsc.store_scatter(buf, [rowv, cols], val16)

        def out_slice(i):
            return out_hbm.at[pl.ds(wid * rows_w + i * _R, _R)]

        def body(i, carry):
            for parity, buf, sem in ((0, buf_a, sem_a), (1, buf_b, sem_b)):

                @pl.when(lax.rem(i, 2) == parity)
                def _():
                    @pl.when(i >= 2)
                    def _():
                        pltpu.make_async_copy(buf, out_slice(i - 2), sem).wait()
                        scatter(buf, (i - 2) * _CW, zeros16)

                    scatter(buf, i * _CW, ones16)
                    pltpu.async_copy(buf, out_slice(i), sem)

            return carry

        lax.fori_loop(0, chunks, body, 0)
        pltpu.make_async_copy(buf_a, out_slice(chunks - 2), sem_a).wait()
        pltpu.make_async_copy(buf_b, out_slice(chunks - 1), sem_b).wait()

    return k


def kernel(inputs):
    x = inputs.astype(jnp.int32).reshape(-1)
    return _make_sc_kernel()(x)


# speedup vs baseline: 4.7086x; 1.0287x over previous
import functools

import jax
import jax.numpy as jnp
from jax import lax
from jax.experimental import pallas as pl
from jax.experimental.pallas import tpu as pltpu
from jax.experimental.pallas import tpu_sc as plsc

_B, _D, _C = 4096, 100, 128
_R = 4
_OFFS = (0, 16, 32, 48, 64, 80, 84)
_CW = _R * _D


def _make_sc_kernel():
    info = plsc.get_sparse_core_info()
    nc, ns = info.num_cores, info.num_subcores
    nw = nc * ns
    rows_w = _B // nw
    chunks = rows_w // _R
    mesh = plsc.VectorSubcoreMesh(core_axis_name="c", subcore_axis_name="s")

    @functools.partial(
        pl.kernel,
        mesh=mesh,
        out_type=jax.ShapeDtypeStruct((_B, _D * _C), jnp.float32),
        scratch_types=[
            pltpu.VMEM((rows_w, _D), jnp.int32),
            pltpu.VMEM((_R, _D * _C), jnp.float32),
            pltpu.VMEM((_R, _D * _C), jnp.float32),
            pltpu.SemaphoreType.DMA,
            pltpu.SemaphoreType.DMA,
        ],
        compiler_params=pltpu.CompilerParams(needs_layout_passes=False),
    )
    def k(x_hbm, out_hbm, idx_v, buf_a, buf_b, sem_a, sem_b):
        wid = lax.axis_index("s") * nc + lax.axis_index("c")
        zeros16 = jnp.zeros((16,), jnp.float32)
        ones16 = jnp.ones((16,), jnp.float32)
        lane = lax.broadcasted_iota(jnp.int32, (16,), 0) * _C

        pltpu.sync_copy(x_hbm.at[pl.ds(wid * rows_w, rows_w)], idx_v)

        def zbody(i, carry):
            base = i * 64
            for r in range(_R):
                for u in range(4):
                    buf_a[r, pl.ds(base + u * 16, 16)] = zeros16
                    buf_b[r, pl.ds(base + u * 16, 16)] = zeros16
            return carry

        lax.fori_loop(0, _D * _C // 64, zbody, 0)

        def scatter(buf, chunk, val16):
            for r in range(_R):
                rowv = jnp.full((16,), r, jnp.int32)
                for off in _OFFS:
                    vals = idx_v[chunk * _R + r, pl.ds(off, 16)]
                    cols = vals + lane + off * _C
                    plsc.store_scatter(buf, [rowv, cols], val16)

        def out_slice(i):
            return out_hbm.at[pl.ds(wid * rows_w + i * _R, _R)]

        def body(i, carry):
            for parity, buf, sem in ((0, buf_a, sem_a), (1, buf_b, sem_b)):

                @pl.when(lax.rem(i, 2) == parity)
                def _():
                    @pl.when(i >= 2)
                    def _():
                        pltpu.make_async_copy(buf, out_slice(i - 2), sem).wait()
                        scatter(buf, i - 2, zeros16)

                    scatter(buf, i, ones16)
                    pltpu.async_copy(buf, out_slice(i), sem)

            return carry

        lax.fori_loop(0, chunks, body, 0)
        pltpu.make_async_copy(buf_a, out_slice(chunks - 2), sem_a).wait()
        pltpu.make_async_copy(buf_b, out_slice(chunks - 1), sem_b).wait()

    return k


def kernel(inputs):
    return _make_sc_kernel()(inputs.astype(jnp.int32))
